# baseline (speedup 1.0000x reference)
import functools

import jax
import jax.numpy as jnp
from jax import lax
from jax.experimental import pallas as pl
from jax.experimental.pallas import tpu as pltpu
from jax.experimental.pallas import tpu_sc as plsc

N_NODES = 10000
N_EDGES = 320000
D = 128
HALF = 64
WSC = 72
CNT_COL = 64
G = 128
NG = N_EDGES // G
NC = 2
NS = 16
GROUPS_PER_SUB = -(-NG // NS)
NPAD = 10240
ROWS_PER_SUB = NPAD // NS


def _sc_segment_sum(nbr_feat, idx2d):
    mesh = plsc.VectorSubcoreMesh(core_axis_name="c", subcore_axis_name="s")

    @functools.partial(
        pl.kernel,
        out_type=jax.ShapeDtypeStruct((NC, NPAD, D), jnp.float32),
        mesh=mesh,
        compiler_params=pltpu.CompilerParams(use_tc_tiling_on_sc=False),
        scratch_types=[
            pltpu.VMEM((2, G), jnp.int32),
            pltpu.VMEM((2, G, WSC), jnp.float32),
            pltpu.VMEM_SHARED((NPAD, WSC), jnp.float32),
            pltpu.SemaphoreType.DMA,
            pltpu.SemaphoreType.DMA,
            pltpu.SemaphoreType.DMA,
        ],
    )
    def k(nbr_hbm, idx_hbm, psum_hbm, idx_v, row_v, acc_sh, sem0, sem1, rsem):
        c = lax.axis_index("c")
        s = lax.axis_index("s")
        zeros16 = jnp.zeros((16,), jnp.float32)
        sems = (sem0, sem1)

        def z_row(r, carry):
            for kk in (0, 16, 32, 48, 56):
                row_v[0, r, pl.ds(kk, 16)] = zeros16
                row_v[1, r, pl.ds(kk, 16)] = zeros16
            return carry
        lax.fori_loop(0, G, z_row, None)

        base = s * ROWS_PER_SUB
        for j in range(ROWS_PER_SUB // G):
            pltpu.sync_copy(row_v.at[0], acc_sh.at[pl.ds(base + j * G, G)])

        e8 = jnp.where(lax.iota(jnp.int32, 16) == 8, 1.0, 0.0)

        def o_row(r, carry):
            row_v[0, r, pl.ds(CNT_COL - 8, 16)] = e8
            row_v[1, r, pl.ds(CNT_COL - 8, 16)] = e8
            return carry
        lax.fori_loop(0, G, o_row, None)
        plsc.subcore_barrier()

        def pred(t):
            return (t < GROUPS_PER_SUB) & (s * GROUPS_PER_SUB + t < NG)

        def start(t, b):
            @pl.when(pred(t))
            def _():
                gid = s * GROUPS_PER_SUB + t
                pltpu.async_copy(idx_hbm.at[gid], idx_v.at[b], sems[b])

                @pl.when(c == 0)
                def _():
                    pltpu.async_copy(
                        nbr_hbm.at[pl.ds(gid * G, G), pl.ds(0, HALF)],
                        row_v.at[b].at[:, pl.ds(0, HALF)], sems[b])

                @pl.when(c == 1)
                def _():
                    pltpu.async_copy(
                        nbr_hbm.at[pl.ds(gid * G, G), pl.ds(HALF, HALF)],
                        row_v.at[b].at[:, pl.ds(0, HALF)], sems[b])

        def proc(t, b):
            @pl.when(pred(t))
            def _():
                pltpu.make_async_copy(idx_hbm.at[0], idx_v.at[b],
                                      sems[b]).wait()
                pltpu.make_async_copy(
                    nbr_hbm.at[pl.ds(0, G), pl.ds(0, HALF)],
                    row_v.at[b].at[:, pl.ds(0, HALF)], sems[b]).wait()
                pltpu.async_copy(row_v.at[b], acc_sh.at[idx_v.at[b]],
                                 rsem, add=True).wait()

        start(0, 0)

        def pair(p, carry):
            t0 = 2 * p
            start(t0 + 1, 1)
            proc(t0, 0)
            start(t0 + 2, 0)
            proc(t0 + 1, 1)
            return carry
        lax.fori_loop(0, (GROUPS_PER_SUB + 1) // 2, pair, None)
        plsc.subcore_barrier()

        pltpu.sync_copy(acc_sh.at[pl.ds(base, ROWS_PER_SUB)],
                        psum_hbm.at[c, pl.ds(base, ROWS_PER_SUB),
                                    pl.ds(0, WSC)])

    return k(nbr_feat, idx2d)


def _tc_body(self_ref, psum_ref, w_ref, o_ref):
    lo = psum_ref[0, :, 0:HALF]
    hi = psum_ref[1, :, 0:HALF]
    cnt = psum_ref[0, :, CNT_COL:CNT_COL + 1]
    inv = 1.0 / jnp.maximum(cnt, 1.0)
    o_ref[...] = (
        jnp.dot(self_ref[...], w_ref[0:D, :],
                preferred_element_type=jnp.float32,
                precision=lax.Precision.HIGHEST)
        + jnp.dot(lo * inv, w_ref[D:D + HALF, :],
                  preferred_element_type=jnp.float32,
                  precision=lax.Precision.HIGHEST)
        + jnp.dot(hi * inv, w_ref[D + HALF:2 * D, :],
                  preferred_element_type=jnp.float32,
                  precision=lax.Precision.HIGHEST)
    )


def _tc_epilogue(self_feat, psum, W):
    B = 1000
    grid = (N_NODES // B,)
    return pl.pallas_call(
        _tc_body,
        grid=grid,
        in_specs=[
            pl.BlockSpec((B, D), lambda i: (i, 0)),
            pl.BlockSpec((NC, B, D), lambda i: (0, i, 0)),
            pl.BlockSpec((2 * D, D), lambda i: (0, 0)),
        ],
        out_specs=pl.BlockSpec((B, D), lambda i: (i, 0)),
        out_shape=jax.ShapeDtypeStruct((N_NODES, D), jnp.float32),
    )(self_feat, psum, W)


def kernel(self_feat, nbr_feat, relation_src_indices, W):
    idx2d = relation_src_indices.astype(jnp.int32).reshape(NG, G)
    psum = _sc_segment_sum(nbr_feat, idx2d)
    out = _tc_epilogue(self_feat, psum, W)
    return out
